# baseline (speedup 1.0000x reference)
import jax
import jax.numpy as jnp
from jax.experimental import pallas as pl
from jax.experimental.pallas import tpu as pltpu

_F32 = jnp.float32
_BF16 = jnp.bfloat16

_B = 4
_T = 168
_N = 2048
_L = 64
_M = _N + _T
_MP = 2304
_BM = 2304


def _elu(v):
    return jnp.where(v > 0, v, jnp.exp(jnp.minimum(v, 0.0)) - 1.0)


def _sigmoid(v):
    return 0.5 * jnp.tanh(0.5 * v) + 0.5


def _fused_body(x_ref, marks_ref, se_ref, w1x_ref, w1s_ref, b1_ref, w2_ref,
                b2_ref, tm_ref, wxr_ref, wxz_ref, wxn_ref, wmr_ref,
                wmz_ref, wmn_ref, ter_ref, tez_ref, ten_ref,
                whr_ref, whz_ref, whn_ref, bhn_ref,
                gw1_ref, gb1_ref, gw2_ref, gb2_ref,
                o_ref, xbf_sc, h1bf_sc, xt_sc, *gi_sc):
    b = pl.program_id(0)

    @pl.when(b == 0)
    def _():
        seproj = (jnp.dot(se_ref[...], w1s_ref[...],
                          preferred_element_type=_F32) + b1_ref[...])
        w2 = w2_ref[...]
        b2 = b2_ref[...]
        ones_col = jnp.ones((_MP, 1), _BF16)
        for bb in range(_B):
            xb = x_ref[bb]
            h = jax.lax.dot_general(xb, w1x_ref[...],
                                    (((0,), (0,)), ((), ())),
                                    preferred_element_type=_F32)
            h = _elu(h + seproj)
            xbf_sc[bb * _MP:bb * _MP + _N, 0:_L] = (
                jnp.dot(h, w2, preferred_element_type=_F32) + b2
            ).astype(_BF16)
            xbf_sc[bb * _MP + _M:(bb + 1) * _MP, 0:_L] = (
                jnp.zeros((_MP - _M, _L), _BF16))
            xbf_sc[bb * _MP:(bb + 1) * _MP, _L:_L + 1] = ones_col

        wx = (wxr_ref, wxz_ref, wxn_ref)
        wm = (wmr_ref, wmz_ref, wmn_ref)
        te = (ter_ref, tez_ref, ten_ref)
        for g in range(3):
            teproj = jnp.dot(tm_ref[...], te[g][...],
                             preferred_element_type=_F32)
            for bb in range(_B):
                gi = jnp.dot(x_ref[bb], wx[g][...],
                             preferred_element_type=_F32)
                gi = gi + jnp.dot(marks_ref[bb], wm[g][...],
                                  preferred_element_type=_F32)
                gi_sc[g * _B + bb][...] = gi + teproj

        whr = whr_ref[...]
        whz = whz_ref[...]
        whn = whn_ref[...]
        bhn = bhn_ref[...]

        def step(t, hc):
            gir = jnp.concatenate(
                [gi_sc[0 * _B + bb][pl.ds(t, 1), :] for bb in range(_B)],
                axis=0)
            giz = jnp.concatenate(
                [gi_sc[1 * _B + bb][pl.ds(t, 1), :] for bb in range(_B)],
                axis=0)
            gin = jnp.concatenate(
                [gi_sc[2 * _B + bb][pl.ds(t, 1), :] for bb in range(_B)],
                axis=0)
            ghr = jnp.dot(hc, whr, preferred_element_type=_F32)
            ghz = jnp.dot(hc, whz, preferred_element_type=_F32)
            ghn = jnp.dot(hc, whn, preferred_element_type=_F32)
            r = _sigmoid(gir + ghr)
            z = _sigmoid(giz + ghz)
            n = jnp.tanh(gin + r * (ghn + bhn))
            hnew = n + z * (hc - n)
            for bb in range(_B):
                xt_sc[pl.ds(bb * _T + t, 1), :] = hnew[bb:bb + 1]
            return hnew

        jax.lax.fori_loop(0, _T, step, jnp.zeros((_B, _L), _F32), unroll=8)
        for bb in range(_B):
            xbf_sc[bb * _MP + _N:bb * _MP + _M, 0:_L] = (
                xt_sc[bb * _T:(bb + 1) * _T, :].astype(_BF16))

    xv = xbf_sc[pl.ds(b * _MP, _MP), 0:_L]
    s = jax.lax.dot_general(xv, xv, (((1,), (1,)), ((), ())),
                            preferred_element_type=_F32)
    abf = jnp.tanh(jnp.maximum(s, 0.0)).astype(_BF16)
    p_aug = jnp.dot(abf, xbf_sc[pl.ds(b * _MP, _MP), :],
                    preferred_element_type=_F32)
    dinv = 1.0 / (p_aug[:, _L:_L + 1] + 1e-6)
    h1 = _elu(jnp.dot(p_aug[:, 0:_L] * dinv, gw1_ref[...],
                      preferred_element_type=_F32) + gb1_ref[...])
    h1bf_sc[:, 0:_L] = h1.astype(_BF16)
    h1bf_sc[:, _L:_L + 1] = jnp.ones((_MP, 1), _BF16)
    p_aug2 = jnp.dot(abf, h1bf_sc[...], preferred_element_type=_F32)
    dinv2 = 1.0 / (p_aug2[:, _L:_L + 1] + 1e-6)
    o_ref[0] = (jnp.dot(p_aug2[:, 0:_L] * dinv2, gw2_ref[...],
                        preferred_element_type=_F32) + gb2_ref[...])


def kernel(x, x_enc_mark, sp_embed, sp_W1, sp_b1, sp_W2, sp_b2, tm_embed,
           gru_Wih, gru_Whh, gru_bih, gru_bhh, gcn_W1, gcn_b1, gcn_W2,
           gcn_b2):
    L = _L
    w1x = sp_W1[:_T]
    w1s = sp_W1[_T:]
    wih_t_full = gru_Wih.T
    whh_t = gru_Whh.T
    wxg = [wih_t_full[:_N, g * L:(g + 1) * L] for g in range(3)]
    wmg = [wih_t_full[_N:_N + 4, g * L:(g + 1) * L] for g in range(3)]
    wtg = [wih_t_full[_N + 4:, g * L:(g + 1) * L] for g in range(3)]
    whg = [whh_t[:, g * L:(g + 1) * L] for g in range(3)]
    tm_aug = jnp.concatenate(
        [tm_embed, jnp.ones((_T, 1), _F32)], axis=1)
    bihg = [gru_bih[g * L:(g + 1) * L] for g in range(3)]
    bhhg = [gru_bhh[g * L:(g + 1) * L] for g in range(3)]
    ter = jnp.concatenate([wtg[0], (bihg[0] + bhhg[0]).reshape(1, L)], axis=0)
    tez = jnp.concatenate([wtg[1], (bihg[1] + bhhg[1]).reshape(1, L)], axis=0)
    ten = jnp.concatenate([wtg[2], bihg[2].reshape(1, L)], axis=0)

    cspec = lambda *shape: pl.BlockSpec(shape, lambda bb: (0,) * len(shape))

    H2 = pl.pallas_call(
        _fused_body,
        grid=(_B,),
        in_specs=[
            cspec(_B, _T, _N),
            cspec(_B, _T, 4),
            cspec(_N, 32),
            cspec(_T, L),
            cspec(32, L),
            cspec(1, L),
            cspec(L, L),
            cspec(1, L),
            cspec(_T, 33),
            cspec(_N, L), cspec(_N, L), cspec(_N, L),
            cspec(4, L), cspec(4, L), cspec(4, L),
            cspec(33, L), cspec(33, L), cspec(33, L),
            cspec(L, L), cspec(L, L), cspec(L, L),
            cspec(1, L),
            cspec(L, L),
            cspec(1, L),
            cspec(L, L),
            cspec(1, L),
        ],
        out_specs=pl.BlockSpec((1, _BM, L), lambda bb: (bb, 0, 0)),
        out_shape=jax.ShapeDtypeStruct((_B, _M, L), _F32),
        scratch_shapes=(
            [pltpu.VMEM((_B * _MP, _L + 1), _BF16),
             pltpu.VMEM((_MP, _L + 1), _BF16),
             pltpu.VMEM((_B * _T, L), _F32)]
            + [pltpu.VMEM((_T, L), _F32)] * 12
        ),
    )(x, x_enc_mark, sp_embed, w1x, w1s, sp_b1.reshape(1, L), sp_W2,
      sp_b2.reshape(1, L), tm_aug, wxg[0], wxg[1], wxg[2],
      wmg[0], wmg[1], wmg[2], ter, tez, ten,
      whg[0], whg[1], whg[2], bhhg[2].reshape(1, L),
      gcn_W1, gcn_b1.reshape(1, L), gcn_W2, gcn_b2.reshape(1, L))
    return H2


# speedup vs baseline: 1.2051x; 1.2051x over previous
import jax
import jax.numpy as jnp
from jax.experimental import pallas as pl
from jax.experimental.pallas import tpu as pltpu

_F32 = jnp.float32
_BF16 = jnp.bfloat16

_B = 4
_T = 168
_N = 2048
_L = 64
_M = _N + _T
_MP = 2304
_BM = 2304


def _elu(v):
    return jnp.where(v > 0, v, jnp.exp(jnp.minimum(v, 0.0)) - 1.0)


def _sigmoid(v):
    return 0.5 * jnp.tanh(0.5 * v) + 0.5


def _fused_body(x_ref, marks_ref, se_ref, w1x_ref, w1s_ref, b1_ref, w2_ref,
                b2_ref, tm_ref, wxr_ref, wxz_ref, wxn_ref, wmr_ref,
                wmz_ref, wmn_ref, ter_ref, tez_ref, ten_ref,
                whr_ref, whz_ref, whn_ref, bhn_ref,
                gw1_ref, gb1_ref, gw2_ref, gb2_ref,
                o_ref, xbf_sc, h1bf_sc, a_sc, xt_sc, *gi_sc):
    b = pl.program_id(0)
    l = pl.program_id(1)

    @pl.when(jnp.logical_and(l == 0, b == 0))
    def _():
        seproj = (jnp.dot(se_ref[...], w1s_ref[...],
                          preferred_element_type=_F32) + b1_ref[...])
        w2 = w2_ref[...]
        b2 = b2_ref[...]
        ones_col = jnp.ones((_MP, 1), _BF16)
        for bb in range(_B):
            xb = x_ref[bb]
            h = jax.lax.dot_general(xb, w1x_ref[...],
                                    (((0,), (0,)), ((), ())),
                                    preferred_element_type=_F32)
            h = _elu(h + seproj)
            xbf_sc[bb * _MP:bb * _MP + _N, 0:_L] = (
                jnp.dot(h, w2, preferred_element_type=_F32) + b2
            ).astype(_BF16)
            xbf_sc[bb * _MP + _M:(bb + 1) * _MP, 0:_L] = (
                jnp.zeros((_MP - _M, _L), _BF16))
            xbf_sc[bb * _MP:(bb + 1) * _MP, _L:_L + 1] = ones_col

        wx = (wxr_ref, wxz_ref, wxn_ref)
        wm = (wmr_ref, wmz_ref, wmn_ref)
        te = (ter_ref, tez_ref, ten_ref)
        for g in range(3):
            teproj = jnp.dot(tm_ref[...], te[g][...],
                             preferred_element_type=_F32)
            for bb in range(_B):
                gi = jnp.dot(x_ref[bb], wx[g][...],
                             preferred_element_type=_F32)
                gi = gi + jnp.dot(marks_ref[bb], wm[g][...],
                                  preferred_element_type=_F32)
                gi_sc[g * _B + bb][...] = gi + teproj

        whr = whr_ref[...]
        whz = whz_ref[...]
        whn = whn_ref[...]
        bhn = bhn_ref[...]

        def step(t, hc):
            gir = jnp.concatenate(
                [gi_sc[0 * _B + bb][pl.ds(t, 1), :] for bb in range(_B)],
                axis=0)
            giz = jnp.concatenate(
                [gi_sc[1 * _B + bb][pl.ds(t, 1), :] for bb in range(_B)],
                axis=0)
            gin = jnp.concatenate(
                [gi_sc[2 * _B + bb][pl.ds(t, 1), :] for bb in range(_B)],
                axis=0)
            ghr = jnp.dot(hc, whr, preferred_element_type=_F32)
            ghz = jnp.dot(hc, whz, preferred_element_type=_F32)
            ghn = jnp.dot(hc, whn, preferred_element_type=_F32)
            r = _sigmoid(gir + ghr)
            z = _sigmoid(giz + ghz)
            n = jnp.tanh(gin + r * (ghn + bhn))
            hnew = n + z * (hc - n)
            for bb in range(_B):
                xt_sc[pl.ds(bb * _T + t, 1), :] = hnew[bb:bb + 1]
            return hnew

        jax.lax.fori_loop(0, _T, step, jnp.zeros((_B, _L), _F32), unroll=8)
        for bb in range(_B):
            xbf_sc[bb * _MP + _N:bb * _MP + _M, 0:_L] = (
                xt_sc[bb * _T:(bb + 1) * _T, :].astype(_BF16))

    @pl.when(l == 0)
    def _():
        xv = xbf_sc[pl.ds(b * _MP, _MP), 0:_L]
        s = jax.lax.dot_general(xv, xv, (((1,), (1,)), ((), ())),
                                preferred_element_type=_F32)
        abf = jnp.tanh(jnp.maximum(s, 0.0)).astype(_BF16)
        a_sc[...] = abf
        p_aug = jnp.dot(abf, xbf_sc[pl.ds(b * _MP, _MP), :],
                        preferred_element_type=_F32)
        dinv = 1.0 / (p_aug[:, _L:_L + 1] + 1e-6)
        h1 = _elu(jnp.dot(p_aug[:, 0:_L] * dinv, gw1_ref[...],
                          preferred_element_type=_F32) + gb1_ref[...])
        h1bf_sc[:, 0:_L] = h1.astype(_BF16)
        h1bf_sc[:, _L:_L + 1] = jnp.ones((_MP, 1), _BF16)
        o_ref[0] = h1

    @pl.when(l == 1)
    def _():
        p_aug = jnp.dot(a_sc[...], h1bf_sc[...],
                        preferred_element_type=_F32)
        dinv = 1.0 / (p_aug[:, _L:_L + 1] + 1e-6)
        o_ref[0] = (jnp.dot(p_aug[:, 0:_L] * dinv, gw2_ref[...],
                            preferred_element_type=_F32) + gb2_ref[...])


def kernel(x, x_enc_mark, sp_embed, sp_W1, sp_b1, sp_W2, sp_b2, tm_embed,
           gru_Wih, gru_Whh, gru_bih, gru_bhh, gcn_W1, gcn_b1, gcn_W2,
           gcn_b2):
    L = _L
    w1x = sp_W1[:_T]
    w1s = sp_W1[_T:]
    wih_t_full = gru_Wih.T
    whh_t = gru_Whh.T
    wxg = [wih_t_full[:_N, g * L:(g + 1) * L] for g in range(3)]
    wmg = [wih_t_full[_N:_N + 4, g * L:(g + 1) * L] for g in range(3)]
    wtg = [wih_t_full[_N + 4:, g * L:(g + 1) * L] for g in range(3)]
    whg = [whh_t[:, g * L:(g + 1) * L] for g in range(3)]
    tm_aug = jnp.concatenate(
        [tm_embed, jnp.ones((_T, 1), _F32)], axis=1)
    bihg = [gru_bih[g * L:(g + 1) * L] for g in range(3)]
    bhhg = [gru_bhh[g * L:(g + 1) * L] for g in range(3)]
    ter = jnp.concatenate([wtg[0], (bihg[0] + bhhg[0]).reshape(1, L)], axis=0)
    tez = jnp.concatenate([wtg[1], (bihg[1] + bhhg[1]).reshape(1, L)], axis=0)
    ten = jnp.concatenate([wtg[2], bihg[2].reshape(1, L)], axis=0)

    cspec = lambda *shape: pl.BlockSpec(shape, lambda bb, l: (0,) * len(shape))

    H2 = pl.pallas_call(
        _fused_body,
        grid=(_B, 2),
        in_specs=[
            cspec(_B, _T, _N),
            cspec(_B, _T, 4),
            cspec(_N, 32),
            cspec(_T, L),
            cspec(32, L),
            cspec(1, L),
            cspec(L, L),
            cspec(1, L),
            cspec(_T, 33),
            cspec(_N, L), cspec(_N, L), cspec(_N, L),
            cspec(4, L), cspec(4, L), cspec(4, L),
            cspec(33, L), cspec(33, L), cspec(33, L),
            cspec(L, L), cspec(L, L), cspec(L, L),
            cspec(1, L),
            cspec(L, L),
            cspec(1, L),
            cspec(L, L),
            cspec(1, L),
        ],
        out_specs=pl.BlockSpec((1, _BM, L), lambda bb, l: (bb, 0, 0)),
        out_shape=jax.ShapeDtypeStruct((_B, _M, L), _F32),
        scratch_shapes=(
            [pltpu.VMEM((_B * _MP, _L + 1), _BF16),
             pltpu.VMEM((_MP, _L + 1), _BF16),
             pltpu.VMEM((_MP, _MP), _BF16),
             pltpu.VMEM((_B * _T, L), _F32)]
            + [pltpu.VMEM((_T, L), _F32)] * 12
        ),
    )(x, x_enc_mark, sp_embed, w1x, w1s, sp_b1.reshape(1, L), sp_W2,
      sp_b2.reshape(1, L), tm_aug, wxg[0], wxg[1], wxg[2],
      wmg[0], wmg[1], wmg[2], ter, tez, ten,
      whg[0], whg[1], whg[2], bhhg[2].reshape(1, L),
      gcn_W1, gcn_b1.reshape(1, L), gcn_W2, gcn_b2.reshape(1, L))
    return H2
